# baseline (speedup 1.0000x reference)
import functools

import jax
import jax.numpy as jnp
from jax import lax
from jax.experimental import pallas as pl
from jax.experimental.pallas import tpu as pltpu, tpu_sc as plsc

NUM_CORES = 2
NUM_SUBCORES = 16
NUM_WORKERS = NUM_CORES * NUM_SUBCORES

BATCH = 4096
SEQ = 200
DIM = 64
BBLK = BATCH // NUM_WORKERS

_mesh = plsc.VectorSubcoreMesh(
    core_axis_name="c",
    subcore_axis_name="s",
    num_cores=NUM_CORES,
    num_subcores=NUM_SUBCORES,
)


@functools.partial(
    pl.kernel,
    out_type=jax.ShapeDtypeStruct((SEQ, 8, NUM_WORKERS, 8, 128), jnp.float32),
    mesh=_mesh,
    scratch_types=[
        pltpu.VMEM((SEQ, BBLK), jnp.int32),
        pltpu.VMEM((SEQ, BBLK), jnp.int32),
        pltpu.VMEM((2, BBLK, 128), jnp.float32),
        pltpu.VMEM((2, DIM, 128), jnp.float32),
        pltpu.SemaphoreType.DMA,
        pltpu.SemaphoreType.DMA,
    ],
    compiler_params=pltpu.CompilerParams(use_tc_tiling_on_sc=True, needs_layout_passes=False),
)
def _embed_sc(table_hbm, idst_hbm, out_hbm, idx_v, pidx_v, rows_v, t_v, sem_g, sem_s):
    wid = lax.axis_index("s") * NUM_CORES + lax.axis_index("c")

    pltpu.sync_copy(idst_hbm.at[:, pl.ds(wid * BBLK, BBLK)], idx_v)

    @pl.loop(0, SEQ)
    def _mkpairs(j):
        for lg in range(8):
            v = idx_v[j, pl.ds(lg * 16, 16)]
            pidx_v[j, pl.ds(lg * 16, 16)] = v >> 1
            idx_v[j, pl.ds(lg * 16, 16)] = (v & 1) * 64

    def fire_gather(s, b):
        pltpu.async_copy(table_hbm.at[pidx_v.at[s]], rows_v.at[b], sem_g)

    def wait_gather():
        pltpu.make_async_copy(
            table_hbm.at[pidx_v.at[0]], rows_v.at[0], sem_g
        ).wait()

    def fire_stores(s, b):
        for R in range(8):
            pltpu.async_copy(t_v.at[b, pl.ds(R * 8, 8)], out_hbm.at[s, R, wid], sem_s)

    def wait_stores():
        for _ in range(8):
            pltpu.make_async_copy(
                t_v.at[0, pl.ds(0, 8)], out_hbm.at[0, 0, 0], sem_s
            ).wait()

    lanes = lax.iota(jnp.int32, 16)

    rots = [(lanes + k) & 15 for k in range(16)]

    def transpose(s, b):
        @pl.loop(0, 8)
        def _(L):
            lb = L * 16
            lvec = lanes + lb
            halfv = idx_v[s, pl.ds(lb, 16)]
            for cb in (0, 16, 32, 48):
                hcb = halfv + cb
                for k in range(16):
                    cvec = hcb + rots[k]
                    g = plsc.load_gather(rows_v.at[b], [lvec, cvec])
                    dvec = rots[k] + cb
                    plsc.store_scatter(t_v.at[b], [dvec, lvec], g)

    def step(s, b, *, first_pair=False, fire_next=True):
        if fire_next:
            fire_gather(s + 1, 1 - b)
        wait_gather()
        if not first_pair:
            wait_stores()
        transpose(s, b)
        fire_stores(s, b)

    fire_gather(0, 0)
    step(0, 0, first_pair=True)
    step(1, 1, first_pair=True)

    @pl.loop(1, SEQ // 2 - 1)
    def _pair(k):
        s0 = k * 2
        step(s0, 0)
        step(s0 + 1, 1)

    step(SEQ - 2, 0)
    step(SEQ - 1, 1, fire_next=False)

    wait_stores()
    wait_stores()


def kernel(input_ids, weight):
    wpair = weight.reshape(500000, 128)
    ids_t = input_ids.astype(jnp.int32).T
    o5 = _embed_sc(wpair, ids_t)
    return o5.transpose(2, 4, 0, 1, 3).reshape(BATCH, SEQ, DIM)


# speedup vs baseline: 1.3389x; 1.3389x over previous
import functools

import jax
import jax.numpy as jnp
from jax import lax
from jax.experimental import pallas as pl
from jax.experimental.pallas import tpu as pltpu, tpu_sc as plsc

NUM_CORES = 2
NUM_SUBCORES = 16
NUM_WORKERS = NUM_CORES * NUM_SUBCORES

BATCH = 4096
SEQ = 200
DIM = 64
BBLK = BATCH // NUM_WORKERS

_mesh = plsc.VectorSubcoreMesh(
    core_axis_name="c",
    subcore_axis_name="s",
    num_cores=NUM_CORES,
    num_subcores=NUM_SUBCORES,
)


@functools.partial(
    pl.kernel,
    out_type=jax.ShapeDtypeStruct((SEQ, 8, NUM_WORKERS, 8, 128), jnp.float32),
    mesh=_mesh,
    scratch_types=[
        pltpu.VMEM((SEQ, BBLK), jnp.int32),
        pltpu.VMEM((SEQ, BBLK), jnp.int32),
        pltpu.VMEM((2, BBLK, 128), jnp.float32),
        pltpu.VMEM((2, DIM, 128), jnp.float32),
        pltpu.SemaphoreType.DMA,
        pltpu.SemaphoreType.DMA,
    ],
    compiler_params=pltpu.CompilerParams(use_tc_tiling_on_sc=True, needs_layout_passes=False),
)
def _embed_sc(table_hbm, idst_hbm, out_hbm, idx_v, pidx_v, rows_v, t_v, sem_g, sem_s):
    wid = lax.axis_index("s") * NUM_CORES + lax.axis_index("c")

    pltpu.sync_copy(idst_hbm.at[:, pl.ds(wid * BBLK, BBLK)], idx_v)

    @pl.loop(0, SEQ)
    def _mkpairs(j):
        for lg in range(8):
            v = idx_v[j, pl.ds(lg * 16, 16)]
            pidx_v[j, pl.ds(lg * 16, 16)] = v >> 1
            idx_v[j, pl.ds(lg * 16, 16)] = (v & 1) * 64

    def fire_gather(s, b):
        pltpu.async_copy(table_hbm.at[pidx_v.at[s]], rows_v.at[b], sem_g)

    def wait_gather():
        pltpu.make_async_copy(
            table_hbm.at[pidx_v.at[0]], rows_v.at[0], sem_g
        ).wait()

    def fire_stores(s, b):
        for R in range(8):
            pltpu.async_copy(t_v.at[b, pl.ds(R * 8, 8)], out_hbm.at[s, R, wid], sem_s)

    def wait_stores():
        for _ in range(8):
            pltpu.make_async_copy(
                t_v.at[0, pl.ds(0, 8)], out_hbm.at[0, 0, 0], sem_s
            ).wait()

    lanes = lax.iota(jnp.int32, 16)

    rots = [(lanes + k) & 15 for k in range(16)]

    def transpose(s, b):
        @pl.loop(0, 8)
        def _(L):
            lb = L * 16
            lvec = lanes + lb
            halfv = idx_v[s, pl.ds(lb, 16)]
            for cb in (0, 16, 32, 48):
                hcb = halfv + cb
                for k0 in range(0, 16, 4):
                    gs = [
                        plsc.load_gather(rows_v.at[b], [lvec, hcb + rots[k0 + i]])
                        for i in range(4)
                    ]
                    for i in range(4):
                        plsc.store_scatter(
                            t_v.at[b], [rots[k0 + i] + cb, lvec], gs[i]
                        )

    def step(s, b, *, first_pair=False, fire_next=True):
        if fire_next:
            fire_gather(s + 1, 1 - b)
        wait_gather()
        if not first_pair:
            wait_stores()
        transpose(s, b)
        fire_stores(s, b)

    fire_gather(0, 0)
    step(0, 0, first_pair=True)
    step(1, 1, first_pair=True)

    @pl.loop(1, SEQ // 2 - 1)
    def _pair(k):
        s0 = k * 2
        step(s0, 0)
        step(s0 + 1, 1)

    step(SEQ - 2, 0)
    step(SEQ - 1, 1, fire_next=False)

    wait_stores()
    wait_stores()


def kernel(input_ids, weight):
    wpair = weight.reshape(500000, 128)
    ids_t = input_ids.astype(jnp.int32).T
    o5 = _embed_sc(wpair, ids_t)
    return o5.transpose(2, 4, 0, 1, 3).reshape(BATCH, SEQ, DIM)


# speedup vs baseline: 1.9974x; 1.4918x over previous
import functools

import jax
import jax.numpy as jnp
from jax import lax
from jax.experimental import pallas as pl
from jax.experimental.pallas import tpu as pltpu, tpu_sc as plsc

NUM_CORES = 2
NUM_SUBCORES = 16
NUM_WORKERS = NUM_CORES * NUM_SUBCORES

BATCH = 4096
SEQ = 200
DIM = 64
BBLK = BATCH // NUM_WORKERS

_mesh = plsc.VectorSubcoreMesh(
    core_axis_name="c",
    subcore_axis_name="s",
    num_cores=NUM_CORES,
    num_subcores=NUM_SUBCORES,
)


@functools.partial(
    pl.kernel,
    out_type=jax.ShapeDtypeStruct((SEQ, 8, NUM_WORKERS, 8, 128), jnp.float32),
    mesh=_mesh,
    scratch_types=[
        pltpu.VMEM((SEQ, BBLK), jnp.int32),
        pltpu.VMEM((SEQ, BBLK), jnp.int32),
        pltpu.VMEM((2, BBLK, 128), jnp.float32),
        pltpu.VMEM((2, DIM, 128), jnp.float32),
        pltpu.SemaphoreType.DMA,
        pltpu.SemaphoreType.DMA,
    ],
    compiler_params=pltpu.CompilerParams(use_tc_tiling_on_sc=True, needs_layout_passes=False),
)
def _embed_sc(table_hbm, idst_hbm, out_hbm, idx_v, pidx_v, rows_v, t_v, sem_g, sem_s):
    wid = lax.axis_index("s") * NUM_CORES + lax.axis_index("c")

    pltpu.sync_copy(idst_hbm.at[:, pl.ds(wid * BBLK, BBLK)], idx_v)

    @pl.loop(0, SEQ)
    def _mkpairs(j):
        for lg in range(8):
            v = idx_v[j, pl.ds(lg * 16, 16)]
            pidx_v[j, pl.ds(lg * 16, 16)] = v >> 1
            idx_v[j, pl.ds(lg * 16, 16)] = (v & 1) * 64

    def fire_gather(s, b):
        pltpu.async_copy(table_hbm.at[pidx_v.at[s]], rows_v.at[b], sem_g)

    def wait_gather():
        pltpu.make_async_copy(
            table_hbm.at[pidx_v.at[0]], rows_v.at[0], sem_g
        ).wait()

    def fire_stores(s, b):
        for R in range(8):
            pltpu.async_copy(t_v.at[b, pl.ds(R * 8, 8)], out_hbm.at[s, R, wid], sem_s)

    def wait_stores():
        for _ in range(8):
            pltpu.make_async_copy(
                t_v.at[0, pl.ds(0, 8)], out_hbm.at[0, 0, 0], sem_s
            ).wait()

    lanes = lax.iota(jnp.int32, 16)

    rots = [(lanes + k) & 15 for k in range(16)]

    def transpose(s, b):
        @pl.loop(0, 8)
        def _(L):
            lb = L * 16
            lvec = lanes + lb
            halfv = idx_v[s, pl.ds(lb, 16)]
            for cb in (0, 16, 32, 48):
                hcb = halfv + cb
                for k0 in range(0, 16, 4):
                    gs = [
                        plsc.load_gather(rows_v.at[b], [lvec, hcb + rots[k0 + i]])
                        for i in range(4)
                    ]
                    for i in range(4):
                        plsc.store_scatter(
                            t_v.at[b], [rots[k0 + i] + cb, lvec], gs[i]
                        )

    def step(s, b, *, first_pair=False, fire_next=True):
        if fire_next:
            fire_gather(s + 1, 1 - b)
        wait_gather()
        if not first_pair:
            wait_stores()
        transpose(s, b)
        fire_stores(s, b)

    fire_gather(0, 0)
    step(0, 0, first_pair=True)
    step(1, 1, first_pair=True)

    @pl.loop(1, SEQ // 2 - 1)
    def _pair(k):
        s0 = k * 2
        step(s0, 0)
        step(s0 + 1, 1)

    step(SEQ - 2, 0)
    step(SEQ - 1, 1, fire_next=False)

    wait_stores()
    wait_stores()


NCOL_FULL = 7808
NPAIR = 500000


@functools.partial(
    pl.kernel,
    out_type=jax.ShapeDtypeStruct((NPAIR, 128), jnp.float32),
    mesh=_mesh,
    scratch_types=[
        pltpu.VMEM((2, DIM, 128), jnp.float32),
        pltpu.VMEM((2, DIM, 128), jnp.float32),
        pltpu.SemaphoreType.DMA,
        pltpu.SemaphoreType.DMA,
    ],
    compiler_params=pltpu.CompilerParams(use_tc_tiling_on_sc=True, needs_layout_passes=False),
)
def _repack_sc(wt_hbm, wtail_hbm, wl_hbm, slab_v, pout_v, sem_i, sem_o):
    wid = lax.axis_index("s") * NUM_CORES + lax.axis_index("c")
    lanes = lax.iota(jnp.int32, 16)
    rots = [(lanes + k) & 15 for k in range(16)]

    def fire_in(c, b):
        pltpu.async_copy(wt_hbm.at[:, pl.ds(c * 128, 128)], slab_v.at[b], sem_i)

    def wait_in():
        pltpu.make_async_copy(
            wt_hbm.at[:, pl.ds(0, 128)], slab_v.at[0], sem_i
        ).wait()

    def fire_out(c, b):
        pltpu.async_copy(pout_v.at[b], wl_hbm.at[pl.ds(c * 64, 64)], sem_o)

    def wait_out():
        pltpu.make_async_copy(
            pout_v.at[0], wl_hbm.at[pl.ds(0, 64)], sem_o
        ).wait()

    def transpose_slab(b, npb):
        @pl.loop(0, npb)
        def _(PB):
            pb = PB * 16
            pvec = lanes + pb
            xbase = pvec * 2
            for h in (0, 1):
                xvec = xbase + h
                for db in (0, 16, 32, 48):
                    hdb = h * 64 + db
                    for k0 in range(0, 16, 4):
                        gs = [
                            plsc.load_gather(
                                slab_v.at[b], [db + rots[k0 + i], xvec]
                            )
                            for i in range(4)
                        ]
                        for i in range(4):
                            plsc.store_scatter(
                                pout_v.at[b], [pvec, hdb + rots[k0 + i]], gs[i]
                            )

    n_w = 244 + jnp.where(wid < 4, 1, 0)

    def col(k):
        return jnp.where(k < 244, wid + k * NUM_WORKERS, NCOL_FULL + wid)

    fire_in(col(0), 0)

    @pl.loop(0, n_w)
    def _rloop(k):
        b = k & 1

        @pl.when(k + 1 < n_w)
        def _():
            fire_in(col(k + 1), 1 - b)

        wait_in()

        @pl.when(k >= 2)
        def _():
            wait_out()

        transpose_slab(b, 4)
        fire_out(col(k), b)

    wait_out()
    wait_out()

    @pl.when(wid == 4)
    def _():
        pltpu.sync_copy(wtail_hbm, slab_v.at[0])
        transpose_slab(0, 2)
        pltpu.sync_copy(
            pout_v.at[0, pl.ds(0, 32)], wl_hbm.at[pl.ds(7812 * 64, 32)]
        )


def kernel(input_ids, weight):
    wt = weight.T
    wtail = jnp.pad(wt[:, 999936:], ((0, 0), (0, 64)))
    wpair = _repack_sc(wt, wtail)
    ids_t = input_ids.astype(jnp.int32).T
    o5 = _embed_sc(wpair, ids_t)
    return o5.transpose(2, 4, 0, 1, 3).reshape(BATCH, SEQ, DIM)


# speedup vs baseline: 2.0079x; 1.0052x over previous
import functools

import jax
import jax.numpy as jnp
from jax import lax
from jax.experimental import pallas as pl
from jax.experimental.pallas import tpu as pltpu, tpu_sc as plsc

NUM_CORES = 2
NUM_SUBCORES = 16
NUM_WORKERS = NUM_CORES * NUM_SUBCORES

BATCH = 4096
SEQ = 200
DIM = 64
BBLK = BATCH // NUM_WORKERS

_mesh = plsc.VectorSubcoreMesh(
    core_axis_name="c",
    subcore_axis_name="s",
    num_cores=NUM_CORES,
    num_subcores=NUM_SUBCORES,
)


@functools.partial(
    pl.kernel,
    out_type=jax.ShapeDtypeStruct((SEQ, 8, NUM_WORKERS, 8, 128), jnp.float32),
    mesh=_mesh,
    scratch_types=[
        pltpu.VMEM((SEQ, BBLK), jnp.int32),
        pltpu.VMEM((SEQ, BBLK), jnp.int32),
        pltpu.VMEM((3, BBLK, 128), jnp.float32),
        pltpu.VMEM((2, DIM, 128), jnp.float32),
        pltpu.SemaphoreType.DMA,
        pltpu.SemaphoreType.DMA,
    ],
    compiler_params=pltpu.CompilerParams(use_tc_tiling_on_sc=True, needs_layout_passes=False),
)
def _embed_sc(table_hbm, idst_hbm, out_hbm, idx_v, pidx_v, rows_v, t_v, sem_g, sem_s):
    wid = lax.axis_index("s") * NUM_CORES + lax.axis_index("c")

    pltpu.sync_copy(idst_hbm.at[:, pl.ds(wid * BBLK, BBLK)], idx_v)

    @pl.loop(0, SEQ)
    def _mkpairs(j):
        for lg in range(8):
            v = idx_v[j, pl.ds(lg * 16, 16)]
            pidx_v[j, pl.ds(lg * 16, 16)] = v >> 1
            idx_v[j, pl.ds(lg * 16, 16)] = (v & 1) * 64

    def fire_gather(s, b):
        pltpu.async_copy(table_hbm.at[pidx_v.at[s]], rows_v.at[b], sem_g)

    def wait_gather():
        pltpu.make_async_copy(
            table_hbm.at[pidx_v.at[0]], rows_v.at[0], sem_g
        ).wait()

    def fire_stores(s, b):
        for R in range(8):
            pltpu.async_copy(t_v.at[b, pl.ds(R * 8, 8)], out_hbm.at[s, R, wid], sem_s)

    def wait_stores():
        for _ in range(8):
            pltpu.make_async_copy(
                t_v.at[0, pl.ds(0, 8)], out_hbm.at[0, 0, 0], sem_s
            ).wait()

    lanes = lax.iota(jnp.int32, 16)

    rots = [(lanes + k) & 15 for k in range(16)]

    def transpose(s, b, bt):
        @pl.loop(0, 8)
        def _(L):
            lb = L * 16
            lvec = lanes + lb
            halfv = idx_v[s, pl.ds(lb, 16)]
            for cb in (0, 16, 32, 48):
                hcb = halfv + cb
                for k0 in range(0, 16, 4):
                    gs = [
                        plsc.load_gather(rows_v.at[b], [lvec, hcb + rots[k0 + i]])
                        for i in range(4)
                    ]
                    for i in range(4):
                        plsc.store_scatter(
                            t_v.at[bt], [rots[k0 + i] + cb, lvec], gs[i]
                        )

    fire_gather(0, 0)
    fire_gather(1, 1)

    @pl.loop(0, SEQ)
    def _step(s):
        b = lax.rem(s, 3)
        bt = s & 1

        @pl.when(s + 2 < SEQ)
        def _():
            fire_gather(s + 2, lax.rem(s + 2, 3))

        wait_gather()

        @pl.when(s >= 2)
        def _():
            wait_stores()

        transpose(s, b, bt)
        fire_stores(s, bt)

    wait_stores()
    wait_stores()


NCOL_FULL = 7808
NPAIR = 500000


@functools.partial(
    pl.kernel,
    out_type=jax.ShapeDtypeStruct((NPAIR, 128), jnp.float32),
    mesh=_mesh,
    scratch_types=[
        pltpu.VMEM((2, DIM, 128), jnp.float32),
        pltpu.VMEM((2, DIM, 128), jnp.float32),
        pltpu.SemaphoreType.DMA,
        pltpu.SemaphoreType.DMA,
    ],
    compiler_params=pltpu.CompilerParams(use_tc_tiling_on_sc=True, needs_layout_passes=False),
)
def _repack_sc(wt_hbm, wtail_hbm, wl_hbm, slab_v, pout_v, sem_i, sem_o):
    wid = lax.axis_index("s") * NUM_CORES + lax.axis_index("c")
    lanes = lax.iota(jnp.int32, 16)
    rots = [(lanes + k) & 15 for k in range(16)]

    def fire_in(c, b):
        pltpu.async_copy(wt_hbm.at[:, pl.ds(c * 128, 128)], slab_v.at[b], sem_i)

    def wait_in():
        pltpu.make_async_copy(
            wt_hbm.at[:, pl.ds(0, 128)], slab_v.at[0], sem_i
        ).wait()

    def fire_out(c, b):
        pltpu.async_copy(pout_v.at[b], wl_hbm.at[pl.ds(c * 64, 64)], sem_o)

    def wait_out():
        pltpu.make_async_copy(
            pout_v.at[0], wl_hbm.at[pl.ds(0, 64)], sem_o
        ).wait()

    def transpose_slab(b, npb):
        @pl.loop(0, npb)
        def _(PB):
            pb = PB * 16
            pvec = lanes + pb
            xbase = pvec * 2
            for h in (0, 1):
                xvec = xbase + h
                for db in (0, 16, 32, 48):
                    hdb = h * 64 + db
                    for k0 in range(0, 16, 4):
                        gs = [
                            plsc.load_gather(
                                slab_v.at[b], [db + rots[k0 + i], xvec]
                            )
                            for i in range(4)
                        ]
                        for i in range(4):
                            plsc.store_scatter(
                                pout_v.at[b], [pvec, hdb + rots[k0 + i]], gs[i]
                            )

    n_w = 244 + jnp.where(wid < 4, 1, 0)

    def col(k):
        return jnp.where(k < 244, wid + k * NUM_WORKERS, NCOL_FULL + wid)

    fire_in(col(0), 0)

    @pl.loop(0, n_w)
    def _rloop(k):
        b = k & 1

        @pl.when(k + 1 < n_w)
        def _():
            fire_in(col(k + 1), 1 - b)

        wait_in()

        @pl.when(k >= 2)
        def _():
            wait_out()

        transpose_slab(b, 4)
        fire_out(col(k), b)

    wait_out()
    wait_out()

    @pl.when(wid == 4)
    def _():
        pltpu.sync_copy(wtail_hbm, slab_v.at[0])
        transpose_slab(0, 2)
        pltpu.sync_copy(
            pout_v.at[0, pl.ds(0, 32)], wl_hbm.at[pl.ds(7812 * 64, 32)]
        )


def kernel(input_ids, weight):
    wt = weight.T
    wtail = jnp.pad(wt[:, 999936:], ((0, 0), (0, 64)))
    wpair = _repack_sc(wt, wtail)
    ids_t = input_ids.astype(jnp.int32).T
    o5 = _embed_sc(wpair, ids_t)
    return o5.transpose(2, 4, 0, 1, 3).reshape(BATCH, SEQ, DIM)
